# Initial kernel scaffold; baseline (speedup 1.0000x reference)
import jax
import jax.numpy as jnp
from jax.experimental import pallas as pl


def kernel(x, edge_index, batch, W1, b1, W2, b2, fc_W, fc_b):
    raise NotImplementedError("write your pallas kernel here")


# speedup vs baseline: 13.9605x; 13.9605x over previous
import functools

import jax
import jax.numpy as jnp
from jax import lax
from jax.experimental import pallas as pl
from jax.experimental.pallas import tpu as pltpu
from jax.experimental.pallas import tpu_sc as plsc

NN = 10000
NE = 320000
NG = 64
DI = 128
DH = 16

NC = 2
NS = 16
NW = NC * NS
K = 128
CH = 79
EP = NW * CH * K
PADN = NN + 112
RPT = PADN // NS


def _sc_agg(src_g, dst_g, y, compute_deg):
    mesh = plsc.VectorSubcoreMesh(
        core_axis_name="c", subcore_axis_name="s", num_cores=NC,
        num_subcores=NS)

    out_type = [jax.ShapeDtypeStruct((NC, PADN, DH), jnp.float32)]
    scratch = {
        "src_v": pltpu.VMEM((CH, K), jnp.int32),
        "dst_v": pltpu.VMEM((CH, K), jnp.int32),
        "rows_v": pltpu.VMEM((2, K, DH), jnp.float32),
        "stage_v": pltpu.VMEM((RPT, DH), jnp.float32),
        "acc_sh": pltpu.VMEM_SHARED((PADN, DH), jnp.float32),
        "sem": pltpu.SemaphoreType.DMA,
    }
    if compute_deg:
        out_type.append(jax.ShapeDtypeStruct((NC, PADN, DH), jnp.float32))
        scratch["ones_v"] = pltpu.VMEM((K, DH), jnp.float32)
        scratch["deg_sh"] = pltpu.VMEM_SHARED((PADN, DH), jnp.float32)

    def body(src_hbm, dst_hbm, y_hbm, *outs, src_v, dst_v, rows_v,
             stage_v, acc_sh, sem, ones_v=None, deg_sh=None):
        if compute_deg:
            acc_out, deg_out = outs
        else:
            (acc_out,) = outs
        c = lax.axis_index("c")
        s = lax.axis_index("s")
        wid = c * NS + s

        def zrow(i, _):
            stage_v[i, :] = jnp.zeros((DH,), jnp.float32)
            return 0
        lax.fori_loop(0, RPT, zrow, 0)
        pltpu.sync_copy(stage_v, acc_sh.at[pl.ds(s * RPT, RPT)])
        if compute_deg:
            pltpu.sync_copy(stage_v, deg_sh.at[pl.ds(s * RPT, RPT)])

            def orow(i, _):
                ones_v[i, :] = jnp.ones((DH,), jnp.float32)
                return 0
            lax.fori_loop(0, K, orow, 0)

        pltpu.sync_copy(src_hbm.at[wid], src_v)
        pltpu.sync_copy(dst_hbm.at[wid], dst_v)

        plsc.subcore_barrier()

        pltpu.async_copy(y_hbm.at[src_v.at[0]], rows_v.at[0], sem)

        def chunk(j, _):
            pltpu.make_async_copy(
                y_hbm.at[src_v.at[0]], rows_v.at[0], sem).wait()

            @pl.when(j + 1 < CH)
            def _prefetch():
                pltpu.async_copy(
                    y_hbm.at[src_v.at[j + 1]],
                    rows_v.at[lax.rem(j + 1, 2)], sem)

            pltpu.sync_copy(rows_v.at[lax.rem(j, 2)],
                            acc_sh.at[dst_v.at[j]], add=True)
            if compute_deg:
                pltpu.sync_copy(ones_v, deg_sh.at[dst_v.at[j]], add=True)
            return 0
        lax.fori_loop(0, CH, chunk, 0)

        plsc.subcore_barrier()
        sl = pl.ds(s * RPT, RPT)
        pltpu.sync_copy(acc_sh.at[sl], acc_out.at[c, sl])
        if compute_deg:
            pltpu.sync_copy(deg_sh.at[sl], deg_out.at[c, sl])

    f = pl.kernel(body, out_type=out_type, mesh=mesh,
                  scratch_types=scratch,
                  compiler_params=pltpu.CompilerParams(
                      use_tc_tiling_on_sc=False))
    return f(src_g, dst_g, y)


def _tc_in(x, W1):
    def body(x_ref, w_ref, o_ref):
        o_ref[...] = jnp.dot(x_ref[...], w_ref[...],
                             preferred_element_type=jnp.float32)
    return pl.pallas_call(
        body,
        out_shape=jax.ShapeDtypeStruct((NN, DH), jnp.float32),
    )(x, W1)


def _tc_mid(p0, p1, d0, d1, b1, W2):
    def body(p0_ref, p1_ref, d0_ref, d1_ref, b_ref, w_ref, o_ref):
        deg = jnp.maximum(d0_ref[...] + d1_ref[...], 1.0)
        h = (p0_ref[...] + p1_ref[...]) / deg + b_ref[...]
        h = jnp.maximum(h, 0.0)
        o_ref[...] = jnp.dot(h, w_ref[...],
                             preferred_element_type=jnp.float32)
    return pl.pallas_call(
        body,
        out_shape=jax.ShapeDtypeStruct((NN, DH), jnp.float32),
    )(p0, p1, d0, d1, b1, W2)


def _tc_out(p0, p1, d0, d1, b2, batch2, fc_W, fc_b):
    def body(p0_ref, p1_ref, d0_ref, d1_ref, b_ref, bat_ref, w_ref,
             fb_ref, o_ref):
        deg = jnp.maximum(d0_ref[...] + d1_ref[...], 1.0)
        h = (p0_ref[...] + p1_ref[...]) / deg + b_ref[...]
        h = jnp.maximum(h, 0.0)
        gids = lax.broadcasted_iota(jnp.int32, (NN, NG), 1)
        onehot = jnp.where(bat_ref[...] == gids, 1.0, 0.0)
        sums = lax.dot_general(onehot, h, (((0,), (0,)), ((), ())),
                               preferred_element_type=jnp.float32)
        cnts = lax.dot_general(onehot, jnp.ones((NN, DH), jnp.float32),
                               (((0,), (0,)), ((), ())),
                               preferred_element_type=jnp.float32)
        pooled = sums / jnp.maximum(cnts, 1.0)
        o_ref[...] = jnp.dot(pooled, w_ref[...],
                             preferred_element_type=jnp.float32) + fb_ref[...]
    return pl.pallas_call(
        body,
        out_shape=jax.ShapeDtypeStruct((NG, 1), jnp.float32),
    )(p0, p1, d0, d1, b2, batch2, fc_W, fc_b)


def kernel(x, edge_index, batch, W1, b1, W2, b2, fc_W, fc_b):
    src = edge_index[0].astype(jnp.int32)
    dst = edge_index[1].astype(jnp.int32)
    pad = EP - NE
    src_g = jnp.concatenate(
        [src, jnp.zeros((pad,), jnp.int32)]).reshape(NW, CH, K)
    dst_g = jnp.concatenate(
        [dst, jnp.full((pad,), NN, jnp.int32)]).reshape(NW, CH, K)
    batch2 = batch.astype(jnp.int32).reshape(NN, 1)
    b1r = b1.reshape(1, DH)
    b2r = b2.reshape(1, DH)
    fbr = fc_b.reshape(1, 1)

    y1 = _tc_in(x, W1)
    agg1, deg = _sc_agg(src_g, dst_g, y1, compute_deg=True)
    d0 = deg[0, :NN]
    d1 = deg[1, :NN]
    y2 = _tc_mid(agg1[0, :NN], agg1[1, :NN], d0, d1, b1r, W2)
    (agg2,) = _sc_agg(src_g, dst_g, y2, compute_deg=False)
    out = _tc_out(agg2[0, :NN], agg2[1, :NN], d0, d1, b2r, batch2,
                  fc_W, fbr)
    return out


# speedup vs baseline: 21.5477x; 1.5435x over previous
import functools

import jax
import jax.numpy as jnp
from jax import lax
from jax.experimental import pallas as pl
from jax.experimental.pallas import tpu as pltpu
from jax.experimental.pallas import tpu_sc as plsc

NN = 10000
NE = 320000
NG = 64
DI = 128
DH = 16

NC = 2
NS = 16
NW = NC * NS
K = 128
CH = 79
EP = NW * CH * K
PADN = NN + 112
RPT = PADN // NS


def _sc_agg(src_g, dst_g, y, compute_deg):
    mesh = plsc.VectorSubcoreMesh(
        core_axis_name="c", subcore_axis_name="s", num_cores=NC,
        num_subcores=NS)

    out_type = [jax.ShapeDtypeStruct((NC, PADN, DH), jnp.float32)]
    scratch = {
        "src_v": pltpu.VMEM((CH, K), jnp.int32),
        "dst_v": pltpu.VMEM((CH, K), jnp.int32),
        "rows_v": pltpu.VMEM((2, K, DH), jnp.float32),
        "stage_v": pltpu.VMEM((RPT, DH), jnp.float32),
        "acc_sh": pltpu.VMEM_SHARED((PADN, DH), jnp.float32),
        "y_sh": pltpu.VMEM_SHARED((NN, DH), jnp.float32),
        "sem": pltpu.SemaphoreType.DMA,
    }
    if compute_deg:
        out_type.append(jax.ShapeDtypeStruct((NC, PADN, DH), jnp.float32))
        scratch["ones_v"] = pltpu.VMEM((K, DH), jnp.float32)
        scratch["deg_sh"] = pltpu.VMEM_SHARED((PADN, DH), jnp.float32)

    def body(src_hbm, dst_hbm, y_hbm, *outs, src_v, dst_v, rows_v,
             stage_v, acc_sh, y_sh, sem, ones_v=None, deg_sh=None):
        if compute_deg:
            acc_out, deg_out = outs
        else:
            (acc_out,) = outs
        c = lax.axis_index("c")
        s = lax.axis_index("s")
        wid = c * NS + s

        def zrow(i, _):
            stage_v[i, :] = jnp.zeros((DH,), jnp.float32)
            return 0
        lax.fori_loop(0, RPT, zrow, 0)
        pltpu.sync_copy(stage_v, acc_sh.at[pl.ds(s * RPT, RPT)])
        if compute_deg:
            pltpu.sync_copy(stage_v, deg_sh.at[pl.ds(s * RPT, RPT)])

            def orow(i, _):
                ones_v[i, :] = jnp.ones((DH,), jnp.float32)
                return 0
            lax.fori_loop(0, K, orow, 0)

        pltpu.sync_copy(src_hbm.at[wid], src_v)
        pltpu.sync_copy(dst_hbm.at[wid], dst_v)

        @pl.when(s == 0)
        def _stage_y():
            pltpu.sync_copy(y_hbm, y_sh)

        plsc.subcore_barrier()

        pltpu.async_copy(y_sh.at[src_v.at[0]], rows_v.at[0], sem)

        def chunk(j, _):
            pltpu.make_async_copy(
                y_sh.at[src_v.at[0]], rows_v.at[0], sem).wait()

            @pl.when(j + 1 < CH)
            def _prefetch():
                pltpu.async_copy(
                    y_sh.at[src_v.at[j + 1]],
                    rows_v.at[lax.rem(j + 1, 2)], sem)

            pltpu.sync_copy(rows_v.at[lax.rem(j, 2)],
                            acc_sh.at[dst_v.at[j]], add=True)
            if compute_deg:
                pltpu.sync_copy(ones_v, deg_sh.at[dst_v.at[j]], add=True)
            return 0
        lax.fori_loop(0, CH, chunk, 0)

        plsc.subcore_barrier()
        sl = pl.ds(s * RPT, RPT)
        pltpu.sync_copy(acc_sh.at[sl], acc_out.at[c, sl])
        if compute_deg:
            pltpu.sync_copy(deg_sh.at[sl], deg_out.at[c, sl])

    f = pl.kernel(body, out_type=out_type, mesh=mesh,
                  scratch_types=scratch,
                  compiler_params=pltpu.CompilerParams(
                      use_tc_tiling_on_sc=False))
    return f(src_g, dst_g, y)


def _tc_in(x, W1):
    def body(x_ref, w_ref, o_ref):
        o_ref[...] = jnp.dot(x_ref[...], w_ref[...],
                             preferred_element_type=jnp.float32)
    return pl.pallas_call(
        body,
        out_shape=jax.ShapeDtypeStruct((NN, DH), jnp.float32),
    )(x, W1)


def _tc_mid(p0, p1, d0, d1, b1, W2):
    def body(p0_ref, p1_ref, d0_ref, d1_ref, b_ref, w_ref, o_ref):
        deg = jnp.maximum(d0_ref[...] + d1_ref[...], 1.0)
        h = (p0_ref[...] + p1_ref[...]) / deg + b_ref[...]
        h = jnp.maximum(h, 0.0)
        o_ref[...] = jnp.dot(h, w_ref[...],
                             preferred_element_type=jnp.float32)
    return pl.pallas_call(
        body,
        out_shape=jax.ShapeDtypeStruct((NN, DH), jnp.float32),
    )(p0, p1, d0, d1, b1, W2)


def _tc_out(p0, p1, d0, d1, b2, batch2, fc_W, fc_b):
    def body(p0_ref, p1_ref, d0_ref, d1_ref, b_ref, bat_ref, w_ref,
             fb_ref, o_ref):
        deg = jnp.maximum(d0_ref[...] + d1_ref[...], 1.0)
        h = (p0_ref[...] + p1_ref[...]) / deg + b_ref[...]
        h = jnp.maximum(h, 0.0)
        gids = lax.broadcasted_iota(jnp.int32, (NN, NG), 1)
        onehot = jnp.where(bat_ref[...] == gids, 1.0, 0.0)
        sums = lax.dot_general(onehot, h, (((0,), (0,)), ((), ())),
                               preferred_element_type=jnp.float32)
        cnts = lax.dot_general(onehot, jnp.ones((NN, DH), jnp.float32),
                               (((0,), (0,)), ((), ())),
                               preferred_element_type=jnp.float32)
        pooled = sums / jnp.maximum(cnts, 1.0)
        o_ref[...] = jnp.dot(pooled, w_ref[...],
                             preferred_element_type=jnp.float32) + fb_ref[...]
    return pl.pallas_call(
        body,
        out_shape=jax.ShapeDtypeStruct((NG, 1), jnp.float32),
    )(p0, p1, d0, d1, b2, batch2, fc_W, fc_b)


def kernel(x, edge_index, batch, W1, b1, W2, b2, fc_W, fc_b):
    src = edge_index[0].astype(jnp.int32)
    dst = edge_index[1].astype(jnp.int32)
    pad = EP - NE
    src_g = jnp.concatenate(
        [src, jnp.zeros((pad,), jnp.int32)]).reshape(NW, CH, K)
    dst_g = jnp.concatenate(
        [dst, jnp.full((pad,), NN, jnp.int32)]).reshape(NW, CH, K)
    batch2 = batch.astype(jnp.int32).reshape(NN, 1)
    b1r = b1.reshape(1, DH)
    b2r = b2.reshape(1, DH)
    fbr = fc_b.reshape(1, 1)

    y1 = _tc_in(x, W1)
    agg1, deg = _sc_agg(src_g, dst_g, y1, compute_deg=True)
    d0 = deg[0, :NN]
    d1 = deg[1, :NN]
    y2 = _tc_mid(agg1[0, :NN], agg1[1, :NN], d0, d1, b1r, W2)
    (agg2,) = _sc_agg(src_g, dst_g, y2, compute_deg=False)
    out = _tc_out(agg2[0, :NN], agg2[1, :NN], d0, d1, b2r, batch2,
                  fc_W, fbr)
    return out


# speedup vs baseline: 24.0497x; 1.1161x over previous
import functools

import jax
import jax.numpy as jnp
from jax import lax
from jax.experimental import pallas as pl
from jax.experimental.pallas import tpu as pltpu
from jax.experimental.pallas import tpu_sc as plsc

NN = 10000
NE = 320000
NG = 64
DI = 128
DH = 16

NC = 2
NS = 16
NW = NC * NS
K = 128
CH = 79
EP = NW * CH * K
PADN = NN + 112
RPT = PADN // NS


def _sc_agg(src_g, dst_g, y, compute_deg):
    mesh = plsc.VectorSubcoreMesh(
        core_axis_name="c", subcore_axis_name="s", num_cores=NC,
        num_subcores=NS)

    out_type = [jax.ShapeDtypeStruct((NC, PADN, DH), jnp.float32)]
    scratch = {
        "src_v": pltpu.VMEM((CH, K), jnp.int32),
        "dst_v": pltpu.VMEM((CH, K), jnp.int32),
        "rows_v": pltpu.VMEM((2, K, DH), jnp.float32),
        "stage_v": pltpu.VMEM((RPT, DH), jnp.float32),
        "acc_sh": pltpu.VMEM_SHARED((PADN, DH), jnp.float32),
        "y_sh": pltpu.VMEM_SHARED((y.shape[0], DH), jnp.float32),
        "sem": pltpu.SemaphoreType.DMA,
    }
    if compute_deg:
        out_type.append(jax.ShapeDtypeStruct((NC, PADN, DH), jnp.float32))
        scratch["ones_v"] = pltpu.VMEM((K, DH), jnp.float32)
        scratch["deg_sh"] = pltpu.VMEM_SHARED((PADN, DH), jnp.float32)

    def body(src_hbm, dst_hbm, y_hbm, *outs, src_v, dst_v, rows_v,
             stage_v, acc_sh, y_sh, sem, ones_v=None, deg_sh=None):
        if compute_deg:
            acc_out, deg_out = outs
        else:
            (acc_out,) = outs
        c = lax.axis_index("c")
        s = lax.axis_index("s")
        wid = c * NS + s

        def zrow(i, _):
            stage_v[i, :] = jnp.zeros((DH,), jnp.float32)
            return 0
        lax.fori_loop(0, RPT, zrow, 0)
        pltpu.sync_copy(stage_v, acc_sh.at[pl.ds(s * RPT, RPT)])
        if compute_deg:
            pltpu.sync_copy(stage_v, deg_sh.at[pl.ds(s * RPT, RPT)])

            def orow(i, _):
                ones_v[i, :] = jnp.ones((DH,), jnp.float32)
                return 0
            lax.fori_loop(0, K, orow, 0)

        pltpu.sync_copy(src_hbm.at[wid], src_v)
        pltpu.sync_copy(dst_hbm.at[wid], dst_v)

        @pl.when(s == 0)
        def _stage_y():
            pltpu.sync_copy(y_hbm, y_sh)

        plsc.subcore_barrier()

        pltpu.async_copy(y_sh.at[src_v.at[0]], rows_v.at[0], sem)

        def chunk(j, _):
            pltpu.make_async_copy(
                y_sh.at[src_v.at[0]], rows_v.at[0], sem).wait()

            @pl.when(j + 1 < CH)
            def _prefetch():
                pltpu.async_copy(
                    y_sh.at[src_v.at[j + 1]],
                    rows_v.at[lax.rem(j + 1, 2)], sem)

            pltpu.sync_copy(rows_v.at[lax.rem(j, 2)],
                            acc_sh.at[dst_v.at[j]], add=True)
            if compute_deg:
                pltpu.sync_copy(ones_v, deg_sh.at[dst_v.at[j]], add=True)
            return 0
        lax.fori_loop(0, CH, chunk, 0)

        plsc.subcore_barrier()
        sl = pl.ds(s * RPT, RPT)
        pltpu.sync_copy(acc_sh.at[sl], acc_out.at[c, sl])
        if compute_deg:
            pltpu.sync_copy(deg_sh.at[sl], deg_out.at[c, sl])

    f = pl.kernel(body, out_type=out_type, mesh=mesh,
                  scratch_types=scratch,
                  compiler_params=pltpu.CompilerParams(
                      use_tc_tiling_on_sc=False))
    return f(src_g, dst_g, y)


def _tc_in(x, W1):
    def body(x_ref, w_ref, o_ref):
        o_ref[...] = jnp.dot(x_ref[...], w_ref[...],
                             preferred_element_type=jnp.float32)
    return pl.pallas_call(
        body,
        out_shape=jax.ShapeDtypeStruct((NN, DH), jnp.float32),
    )(x, W1)


def _tc_mid(agg, deg, b1, W2):
    def body(a_ref, d_ref, b_ref, w_ref, o_ref):
        dg = jnp.maximum(d_ref[0] + d_ref[1], 1.0)
        h = (a_ref[0] + a_ref[1]) / dg + b_ref[...]
        h = jnp.maximum(h, 0.0)
        o_ref[...] = jnp.dot(h, w_ref[...],
                             preferred_element_type=jnp.float32)
    return pl.pallas_call(
        body,
        out_shape=jax.ShapeDtypeStruct((PADN, DH), jnp.float32),
    )(agg, deg, b1, W2)


def _tc_out(agg, deg, b2, batchp, fc_W, fc_b):
    def body(a_ref, d_ref, b_ref, bat_ref, w_ref, fb_ref, o_ref):
        dg = jnp.maximum(d_ref[0] + d_ref[1], 1.0)
        h = (a_ref[0] + a_ref[1]) / dg + b_ref[...]
        h = jnp.maximum(h, 0.0)
        gids = lax.broadcasted_iota(jnp.int32, (PADN, NG), 1)
        onehot = jnp.where(bat_ref[...] == gids, 1.0, 0.0)
        sums = lax.dot_general(onehot, h, (((0,), (0,)), ((), ())),
                               preferred_element_type=jnp.float32)
        cnts = lax.dot_general(onehot, jnp.ones((PADN, DH), jnp.float32),
                               (((0,), (0,)), ((), ())),
                               preferred_element_type=jnp.float32)
        pooled = sums / jnp.maximum(cnts, 1.0)
        o_ref[...] = jnp.dot(pooled, w_ref[...],
                             preferred_element_type=jnp.float32) + fb_ref[...]
    return pl.pallas_call(
        body,
        out_shape=jax.ShapeDtypeStruct((NG, 1), jnp.float32),
    )(agg, deg, b2, batchp, fc_W, fc_b)


def kernel(x, edge_index, batch, W1, b1, W2, b2, fc_W, fc_b):
    src = edge_index[0].astype(jnp.int32)
    dst = edge_index[1].astype(jnp.int32)
    pad = EP - NE
    src_g = jnp.concatenate(
        [src, jnp.zeros((pad,), jnp.int32)]).reshape(NW, CH, K)
    dst_g = jnp.concatenate(
        [dst, jnp.full((pad,), NN, jnp.int32)]).reshape(NW, CH, K)
    batchp = jnp.concatenate(
        [batch.astype(jnp.int32),
         jnp.full((PADN - NN,), -1, jnp.int32)]).reshape(PADN, 1)
    b1r = b1.reshape(1, DH)
    b2r = b2.reshape(1, DH)
    fbr = fc_b.reshape(1, 1)

    y1 = _tc_in(x, W1)
    agg1, deg = _sc_agg(src_g, dst_g, y1, compute_deg=True)
    y2 = _tc_mid(agg1, deg, b1r, W2)
    (agg2,) = _sc_agg(src_g, dst_g, y2, compute_deg=False)
    out = _tc_out(agg2, deg, b2r, batchp, fc_W, fbr)
    return out


# speedup vs baseline: 32.4132x; 1.3478x over previous
import functools

import jax
import jax.numpy as jnp
from jax import lax
from jax.experimental import pallas as pl
from jax.experimental.pallas import tpu as pltpu
from jax.experimental.pallas import tpu_sc as plsc

NN = 10000
NE = 320000
NG = 64
DI = 128
DH = 16

NC = 2
NS = 16
NW = NC * NS
K = 128
CH = 79
EP = NW * CH * K
PADN = NN + 112
RPT = PADN // NS


def _sc_agg(src_g, dst_g, y, compute_deg):
    mesh = plsc.VectorSubcoreMesh(
        core_axis_name="c", subcore_axis_name="s", num_cores=NC,
        num_subcores=NS)

    out_type = [jax.ShapeDtypeStruct((NC, PADN, DH), jnp.float32)]
    scratch = {
        "src_v": pltpu.VMEM((CH, K), jnp.int32),
        "dst_v": pltpu.VMEM((CH, K), jnp.int32),
        "rows_v": pltpu.VMEM((2, K, DH), jnp.float32),
        "stage_v": pltpu.VMEM((RPT, DH), jnp.float32),
        "acc_sh": pltpu.VMEM_SHARED((PADN, DH), jnp.float32),
        "y_sh": pltpu.VMEM_SHARED((y.shape[0], DH), jnp.float32),
        "sem": pltpu.SemaphoreType.DMA,
    }
    if compute_deg:
        out_type.append(jax.ShapeDtypeStruct((NC, PADN, DH), jnp.float32))
        scratch["ones_v"] = pltpu.VMEM((K, DH), jnp.float32)
        scratch["deg_sh"] = pltpu.VMEM_SHARED((PADN, DH), jnp.float32)

    def body(src_hbm, dst_hbm, y_hbm, *outs, src_v, dst_v, rows_v,
             stage_v, acc_sh, y_sh, sem, ones_v=None, deg_sh=None):
        if compute_deg:
            acc_out, deg_out = outs
        else:
            (acc_out,) = outs
        c = lax.axis_index("c")
        s = lax.axis_index("s")
        wid = c * NS + s

        def zrow(i, _):
            stage_v[i, :] = jnp.zeros((DH,), jnp.float32)
            return 0
        lax.fori_loop(0, RPT, zrow, 0)
        pltpu.sync_copy(stage_v, acc_sh.at[pl.ds(s * RPT, RPT)])
        if compute_deg:
            pltpu.sync_copy(stage_v, deg_sh.at[pl.ds(s * RPT, RPT)])

            def orow(i, _):
                ones_v[i, :] = jnp.ones((DH,), jnp.float32)
                return 0
            lax.fori_loop(0, K, orow, 0)

        pltpu.sync_copy(src_hbm.at[wid], src_v)
        pltpu.sync_copy(dst_hbm.at[wid], dst_v)

        @pl.when(s == 0)
        def _stage_y():
            pltpu.sync_copy(y_hbm, y_sh)

        plsc.subcore_barrier()

        pltpu.async_copy(y_sh.at[src_v.at[0]], rows_v.at[0], sem)

        def chunk(j, _):
            pltpu.make_async_copy(
                y_sh.at[src_v.at[0]], rows_v.at[0], sem).wait()

            @pl.when(j + 1 < CH)
            def _prefetch():
                pltpu.async_copy(
                    y_sh.at[src_v.at[j + 1]],
                    rows_v.at[lax.rem(j + 1, 2)], sem)

            pltpu.sync_copy(rows_v.at[lax.rem(j, 2)],
                            acc_sh.at[dst_v.at[j]], add=True)
            if compute_deg:
                pltpu.sync_copy(ones_v, deg_sh.at[dst_v.at[j]], add=True)
            return 0
        lax.fori_loop(0, CH, chunk, 0)

        plsc.subcore_barrier()
        sl = pl.ds(s * RPT, RPT)
        pltpu.sync_copy(acc_sh.at[sl], acc_out.at[c, sl])
        if compute_deg:
            pltpu.sync_copy(deg_sh.at[sl], deg_out.at[c, sl])

    f = pl.kernel(body, out_type=out_type, mesh=mesh,
                  scratch_types=scratch,
                  compiler_params=pltpu.CompilerParams(
                      use_tc_tiling_on_sc=False))
    return f(src_g, dst_g, y)


def _blockdiag(w_ref):
    w8 = jnp.concatenate([w_ref[...]] * 8, axis=0)
    w88 = jnp.concatenate([w8] * 8, axis=1)
    r = lax.broadcasted_iota(jnp.int32, (8 * DH, 8 * DH), 0) // DH
    c = lax.broadcasted_iota(jnp.int32, (8 * DH, 8 * DH), 1) // DH
    return jnp.where(r == c, w88, 0.0)


def _tile_bias(b_ref):
    return jnp.concatenate([b_ref[...]] * 8, axis=1)


def _tc_in(x, W1):
    def body(x_ref, w_ref, o_ref):
        o_ref[...] = jnp.dot(x_ref[...], w_ref[...],
                             preferred_element_type=jnp.float32)
    return pl.pallas_call(
        body,
        out_shape=jax.ShapeDtypeStruct((NN, DH), jnp.float32),
    )(x, W1)


PADR = PADN // 8


def _tc_mid(agg, deg, b1, W2):
    def body(a_ref, d_ref, b_ref, w_ref, o_ref):
        dg = jnp.maximum(d_ref[0] + d_ref[1], 1.0)
        h = (a_ref[0] + a_ref[1]) / dg + _tile_bias(b_ref)
        h = jnp.maximum(h, 0.0)
        o_ref[...] = jnp.dot(h, _blockdiag(w_ref),
                             preferred_element_type=jnp.float32)
    return pl.pallas_call(
        body,
        out_shape=jax.ShapeDtypeStruct((PADR, 8 * DH), jnp.float32),
    )(agg, deg, b1, W2)


def _tc_out(agg, deg, b2, batchq, fc_W, fc_b):
    def body(a_ref, d_ref, b_ref, bat_ref, w_ref, fb_ref, o_ref):
        dg = jnp.maximum(d_ref[0] + d_ref[1], 1.0)
        h = (a_ref[0] + a_ref[1]) / dg + _tile_bias(b_ref)
        h = jnp.maximum(h, 0.0)
        gids = lax.broadcasted_iota(jnp.int32, (NG, PADR), 0)
        sums = jnp.zeros((NG, DH), jnp.float32)
        cnts = jnp.zeros((NG, DH), jnp.float32)
        ones = jnp.ones((PADR, DH), jnp.float32)
        for q in range(8):
            oh = jnp.where(bat_ref[q:q + 1, :] == gids, 1.0, 0.0)
            hq = h[:, q * DH:(q + 1) * DH]
            sums = sums + lax.dot_general(
                oh, hq, (((1,), (0,)), ((), ())),
                preferred_element_type=jnp.float32)
            cnts = cnts + lax.dot_general(
                oh, ones, (((1,), (0,)), ((), ())),
                preferred_element_type=jnp.float32)
        pooled = sums / jnp.maximum(cnts, 1.0)
        o_ref[...] = jnp.dot(pooled, w_ref[...],
                             preferred_element_type=jnp.float32) + fb_ref[...]
    return pl.pallas_call(
        body,
        out_shape=jax.ShapeDtypeStruct((NG, 1), jnp.float32),
    )(agg, deg, b2, batchq, fc_W, fc_b)


def kernel(x, edge_index, batch, W1, b1, W2, b2, fc_W, fc_b):
    src = edge_index[0].astype(jnp.int32)
    dst = edge_index[1].astype(jnp.int32)
    pad = EP - NE
    src_g = jnp.concatenate(
        [src, jnp.zeros((pad,), jnp.int32)]).reshape(NW, CH, K)
    dst_g = jnp.concatenate(
        [dst, jnp.full((pad,), NN, jnp.int32)]).reshape(NW, CH, K)
    batchq = jnp.transpose(jnp.concatenate(
        [batch.astype(jnp.int32),
         jnp.full((PADN - NN,), -1, jnp.int32)]).reshape(PADR, 8))
    b1r = b1.reshape(1, DH)
    b2r = b2.reshape(1, DH)
    fbr = fc_b.reshape(1, 1)

    y1 = _tc_in(x, W1)
    agg1, deg = _sc_agg(src_g, dst_g, y1, compute_deg=True)
    agg1t = agg1.reshape(NC, PADR, 8 * DH)
    degt = deg.reshape(NC, PADR, 8 * DH)
    y2 = _tc_mid(agg1t, degt, b1r, W2).reshape(PADN, DH)
    (agg2,) = _sc_agg(src_g, dst_g, y2, compute_deg=False)
    out = _tc_out(agg2.reshape(NC, PADR, 8 * DH), degt, b2r, batchq,
                  fc_W, fbr)
    return out


# speedup vs baseline: 36.3104x; 1.1202x over previous
import functools

import jax
import jax.numpy as jnp
from jax import lax
from jax.experimental import pallas as pl
from jax.experimental.pallas import tpu as pltpu
from jax.experimental.pallas import tpu_sc as plsc

NN = 10000
NE = 320000
NG = 64
DI = 128
DH = 16

NC = 2
NS = 16
NW = NC * NS
K = 128
CH = 79
NCHUNK = NE // K
PADN = NN + 112
RPT = PADN // NS


def _sc_agg(ei, y, compute_deg):
    mesh = plsc.VectorSubcoreMesh(
        core_axis_name="c", subcore_axis_name="s", num_cores=NC,
        num_subcores=NS)

    out_type = [jax.ShapeDtypeStruct((NC, PADN, DH), jnp.float32)]
    scratch = {
        "ei_v": pltpu.VMEM((CH, 2, K), jnp.int32),
        "rows_v": pltpu.VMEM((2, K, DH), jnp.float32),
        "stage_v": pltpu.VMEM((RPT, DH), jnp.float32),
        "acc_sh": pltpu.VMEM_SHARED((PADN, DH), jnp.float32),
        "y_sh": pltpu.VMEM_SHARED((PADN, DH), jnp.float32),
        "sem": pltpu.SemaphoreType.DMA,
    }
    if compute_deg:
        out_type.append(jax.ShapeDtypeStruct((NC, PADN, DH), jnp.float32))
        scratch["ones_v"] = pltpu.VMEM((K, DH), jnp.float32)
        scratch["deg_sh"] = pltpu.VMEM_SHARED((PADN, DH), jnp.float32)

    def body(ei_hbm, y_hbm, *outs, ei_v, rows_v,
             stage_v, acc_sh, y_sh, sem, ones_v=None, deg_sh=None):
        if compute_deg:
            acc_out, deg_out = outs
        else:
            (acc_out,) = outs
        c = lax.axis_index("c")
        s = lax.axis_index("s")
        wid = c * NS + s

        def zrow(i, _):
            stage_v[i, :] = jnp.zeros((DH,), jnp.float32)
            return 0
        lax.fori_loop(0, RPT, zrow, 0)
        pltpu.sync_copy(stage_v, acc_sh.at[pl.ds(s * RPT, RPT)])
        if compute_deg:
            pltpu.sync_copy(stage_v, deg_sh.at[pl.ds(s * RPT, RPT)])

            def orow(i, _):
                ones_v[i, :] = jnp.ones((DH,), jnp.float32)
                return 0
            lax.fori_loop(0, K, orow, 0)

        pltpu.sync_copy(ei_hbm.at[pl.ds(wid * CH, CH)], ei_v)

        @pl.when(s == 0)
        def _stage_y():
            pltpu.sync_copy(y_hbm, y_sh.at[pl.ds(0, y_hbm.shape[0])])

        plsc.subcore_barrier()

        pltpu.async_copy(y_sh.at[ei_v.at[0, 0]], rows_v.at[0], sem)

        def chunk(j, _):
            pltpu.make_async_copy(
                y_sh.at[ei_v.at[0, 0]], rows_v.at[0], sem).wait()

            @pl.when(j + 1 < CH)
            def _prefetch():
                pltpu.async_copy(
                    y_sh.at[ei_v.at[j + 1, 0]],
                    rows_v.at[lax.rem(j + 1, 2)], sem)

            pltpu.sync_copy(rows_v.at[lax.rem(j, 2)],
                            acc_sh.at[ei_v.at[j, 1]], add=True)
            if compute_deg:
                pltpu.sync_copy(ones_v, deg_sh.at[ei_v.at[j, 1]], add=True)
            return 0
        lax.fori_loop(0, CH, chunk, 0)

        plsc.subcore_barrier()
        sl = pl.ds(s * RPT, RPT)
        pltpu.sync_copy(acc_sh.at[sl], acc_out.at[c, sl])
        if compute_deg:
            pltpu.sync_copy(deg_sh.at[sl], deg_out.at[c, sl])

    f = pl.kernel(body, out_type=out_type, mesh=mesh,
                  scratch_types=scratch,
                  compiler_params=pltpu.CompilerParams(
                      use_tc_tiling_on_sc=False))
    return f(ei, y)


def _blockdiag(w_ref):
    w8 = jnp.concatenate([w_ref[...]] * 8, axis=0)
    w88 = jnp.concatenate([w8] * 8, axis=1)
    r = lax.broadcasted_iota(jnp.int32, (8 * DH, 8 * DH), 0) // DH
    c = lax.broadcasted_iota(jnp.int32, (8 * DH, 8 * DH), 1) // DH
    return jnp.where(r == c, w88, 0.0)


def _tile_bias(b_ref):
    return jnp.concatenate([b_ref[...]] * 8, axis=1)


def _tc_in(x, W1):
    def body(x_ref, w_ref, o_ref):
        o_ref[...] = jnp.dot(x_ref[...], w_ref[...],
                             preferred_element_type=jnp.float32)
    return pl.pallas_call(
        body,
        out_shape=jax.ShapeDtypeStruct((NN, DH), jnp.float32),
    )(x, W1)


PADR = PADN // 8


def _tc_mid(agg, deg, b1, W2):
    def body(a_ref, d_ref, b_ref, w_ref, o_ref):
        dg = jnp.maximum(d_ref[0] + d_ref[1], 1.0)
        h = (a_ref[0] + a_ref[1]) / dg + _tile_bias(b_ref)
        h = jnp.maximum(h, 0.0)
        o_ref[...] = jnp.dot(h, _blockdiag(w_ref),
                             preferred_element_type=jnp.float32)
    return pl.pallas_call(
        body,
        out_shape=jax.ShapeDtypeStruct((PADR, 8 * DH), jnp.float32),
    )(agg, deg, b1, W2)


def _tc_out(agg, deg, b2, batchq, fc_W, fc_b):
    def body(a_ref, d_ref, b_ref, bat_ref, w_ref, fb_ref, o_ref):
        dg = jnp.maximum(d_ref[0] + d_ref[1], 1.0)
        h = (a_ref[0] + a_ref[1]) / dg + _tile_bias(b_ref)
        h = jnp.maximum(h, 0.0)
        gids = lax.broadcasted_iota(jnp.int32, (NG, PADR), 0)
        sums = jnp.zeros((NG, DH), jnp.float32)
        cnts = jnp.zeros((NG, DH), jnp.float32)
        ones = jnp.ones((PADR, DH), jnp.float32)
        for q in range(8):
            oh = jnp.where(bat_ref[q:q + 1, :] == gids, 1.0, 0.0)
            hq = h[:, q * DH:(q + 1) * DH]
            sums = sums + lax.dot_general(
                oh, hq, (((1,), (0,)), ((), ())),
                preferred_element_type=jnp.float32)
            cnts = cnts + lax.dot_general(
                oh, ones, (((1,), (0,)), ((), ())),
                preferred_element_type=jnp.float32)
        pooled = sums / jnp.maximum(cnts, 1.0)
        o_ref[...] = jnp.dot(pooled, w_ref[...],
                             preferred_element_type=jnp.float32) + fb_ref[...]
    return pl.pallas_call(
        body,
        out_shape=jax.ShapeDtypeStruct((NG, 1), jnp.float32),
    )(agg, deg, b2, batchq, fc_W, fc_b)


def kernel(x, edge_index, batch, W1, b1, W2, b2, fc_W, fc_b):
    ei = jnp.transpose(
        edge_index.astype(jnp.int32).reshape(2, NCHUNK, K), (1, 0, 2))
    ei = jnp.pad(ei, ((0, NW * CH - NCHUNK), (0, 0), (0, 0)),
                 constant_values=NN)
    batchq = jnp.transpose(jnp.concatenate(
        [batch.astype(jnp.int32),
         jnp.full((PADN - NN,), -1, jnp.int32)]).reshape(PADR, 8))
    b1r = b1.reshape(1, DH)
    b2r = b2.reshape(1, DH)
    fbr = fc_b.reshape(1, 1)

    y1 = _tc_in(x, W1)
    agg1, deg = _sc_agg(ei, y1, compute_deg=True)
    agg1t = agg1.reshape(NC, PADR, 8 * DH)
    degt = deg.reshape(NC, PADR, 8 * DH)
    y2 = _tc_mid(agg1t, degt, b1r, W2).reshape(PADN, DH)
    (agg2,) = _sc_agg(ei, y2, compute_deg=False)
    out = _tc_out(agg2.reshape(NC, PADR, 8 * DH), degt, b2r, batchq,
                  fc_W, fbr)
    return out


# speedup vs baseline: 37.4756x; 1.0321x over previous
import functools

import jax
import jax.numpy as jnp
from jax import lax
from jax.experimental import pallas as pl
from jax.experimental.pallas import tpu as pltpu
from jax.experimental.pallas import tpu_sc as plsc

NN = 10000
NE = 320000
NG = 64
DI = 128
DH = 16

NC = 2
NS = 16
NW = NC * NS
K = 128
CH = 79
NCHUNK = NE // K
PADN = NN + 112
RPT = PADN // NS


def _sc_agg(ei, y, compute_deg):
    mesh = plsc.VectorSubcoreMesh(
        core_axis_name="c", subcore_axis_name="s", num_cores=NC,
        num_subcores=NS)

    out_type = [jax.ShapeDtypeStruct((NC, PADN, DH), jnp.float32)]
    scratch = {
        "ei_v": pltpu.VMEM((CH, 2, K), jnp.int32),
        "rows_v": pltpu.VMEM((2, K, DH), jnp.float32),
        "stage_v": pltpu.VMEM((RPT, DH), jnp.float32),
        "acc_sh": pltpu.VMEM_SHARED((PADN, DH), jnp.float32),
        "y_sh": pltpu.VMEM_SHARED((PADN, DH), jnp.float32),
        "sem": pltpu.SemaphoreType.DMA,
        "sem_s": pltpu.SemaphoreType.DMA,
    }
    if compute_deg:
        out_type.append(jax.ShapeDtypeStruct((NC, PADN, DH), jnp.float32))
        scratch["ones_v"] = pltpu.VMEM((K, DH), jnp.float32)
        scratch["deg_sh"] = pltpu.VMEM_SHARED((PADN, DH), jnp.float32)

    def body(ei_hbm, y_hbm, *outs, ei_v, rows_v,
             stage_v, acc_sh, y_sh, sem, sem_s, ones_v=None, deg_sh=None):
        if compute_deg:
            acc_out, deg_out = outs
        else:
            (acc_out,) = outs
        c = lax.axis_index("c")
        s = lax.axis_index("s")
        wid = c * NS + s

        def zrow(i, _):
            stage_v[i, :] = jnp.zeros((DH,), jnp.float32)
            return 0
        lax.fori_loop(0, RPT, zrow, 0)
        pltpu.sync_copy(stage_v, acc_sh.at[pl.ds(s * RPT, RPT)])
        if compute_deg:
            pltpu.sync_copy(stage_v, deg_sh.at[pl.ds(s * RPT, RPT)])

            def orow(i, _):
                ones_v[i, :] = jnp.ones((DH,), jnp.float32)
                return 0
            lax.fori_loop(0, K, orow, 0)

        pltpu.sync_copy(ei_hbm.at[pl.ds(wid * CH, CH)], ei_v)

        @pl.when(s == 0)
        def _stage_y():
            pltpu.sync_copy(y_hbm, y_sh.at[pl.ds(0, y_hbm.shape[0])])

        plsc.subcore_barrier()

        def _wait_sc(j_idx):
            pltpu.make_async_copy(
                rows_v.at[0], acc_sh.at[ei_v.at[j_idx, 1]], sem_s).wait()
            if compute_deg:
                pltpu.make_async_copy(
                    ones_v, deg_sh.at[ei_v.at[j_idx, 1]], sem_s).wait()

        pltpu.async_copy(y_sh.at[ei_v.at[0, 0]], rows_v.at[0], sem)

        def chunk(j, _):
            pltpu.make_async_copy(
                y_sh.at[ei_v.at[0, 0]], rows_v.at[0], sem).wait()
            pltpu.async_copy(rows_v.at[lax.rem(j, 2)],
                             acc_sh.at[ei_v.at[j, 1]], sem_s, add=True)
            if compute_deg:
                pltpu.async_copy(ones_v, deg_sh.at[ei_v.at[j, 1]],
                                 sem_s, add=True)

            @pl.when(j + 1 < CH)
            def _prefetch():
                @pl.when(j >= 1)
                def _drain_prev():
                    _wait_sc(j - 1)
                pltpu.async_copy(
                    y_sh.at[ei_v.at[j + 1, 0]],
                    rows_v.at[lax.rem(j + 1, 2)], sem)
            return 0
        lax.fori_loop(0, CH, chunk, 0)
        _wait_sc(CH - 2)
        _wait_sc(CH - 1)

        plsc.subcore_barrier()
        sl = pl.ds(s * RPT, RPT)
        pltpu.sync_copy(acc_sh.at[sl], acc_out.at[c, sl])
        if compute_deg:
            pltpu.sync_copy(deg_sh.at[sl], deg_out.at[c, sl])

    f = pl.kernel(body, out_type=out_type, mesh=mesh,
                  scratch_types=scratch,
                  compiler_params=pltpu.CompilerParams(
                      use_tc_tiling_on_sc=False))
    return f(ei, y)


def _blockdiag(w_ref):
    w8 = jnp.concatenate([w_ref[...]] * 8, axis=0)
    w88 = jnp.concatenate([w8] * 8, axis=1)
    r = lax.broadcasted_iota(jnp.int32, (8 * DH, 8 * DH), 0) // DH
    c = lax.broadcasted_iota(jnp.int32, (8 * DH, 8 * DH), 1) // DH
    return jnp.where(r == c, w88, 0.0)


def _tile_bias(b_ref):
    return jnp.concatenate([b_ref[...]] * 8, axis=1)


def _tc_in(x, W1):
    def body(x_ref, w_ref, o_ref):
        o_ref[...] = jnp.dot(x_ref[...], w_ref[...],
                             preferred_element_type=jnp.float32)
    return pl.pallas_call(
        body,
        out_shape=jax.ShapeDtypeStruct((NN, DH), jnp.float32),
    )(x, W1)


PADR = PADN // 8


def _tc_mid(agg, deg, b1, W2):
    def body(a_ref, d_ref, b_ref, w_ref, o_ref):
        dg = jnp.maximum(d_ref[0] + d_ref[1], 1.0)
        h = (a_ref[0] + a_ref[1]) / dg + _tile_bias(b_ref)
        h = jnp.maximum(h, 0.0)
        o_ref[...] = jnp.dot(h, _blockdiag(w_ref),
                             preferred_element_type=jnp.float32)
    return pl.pallas_call(
        body,
        out_shape=jax.ShapeDtypeStruct((PADR, 8 * DH), jnp.float32),
    )(agg, deg, b1, W2)


def _tc_out(agg, deg, b2, batchq, fc_W, fc_b):
    def body(a_ref, d_ref, b_ref, bat_ref, w_ref, fb_ref, o_ref):
        dg = jnp.maximum(d_ref[0] + d_ref[1], 1.0)
        h = (a_ref[0] + a_ref[1]) / dg + _tile_bias(b_ref)
        h = jnp.maximum(h, 0.0)
        gids = lax.broadcasted_iota(jnp.int32, (NG, PADR), 0)
        sums = jnp.zeros((NG, DH), jnp.float32)
        cnts = jnp.zeros((NG, DH), jnp.float32)
        ones = jnp.ones((PADR, DH), jnp.float32)
        for q in range(8):
            oh = jnp.where(bat_ref[q:q + 1, :] == gids, 1.0, 0.0)
            hq = h[:, q * DH:(q + 1) * DH]
            sums = sums + lax.dot_general(
                oh, hq, (((1,), (0,)), ((), ())),
                preferred_element_type=jnp.float32)
            cnts = cnts + lax.dot_general(
                oh, ones, (((1,), (0,)), ((), ())),
                preferred_element_type=jnp.float32)
        pooled = sums / jnp.maximum(cnts, 1.0)
        o_ref[...] = jnp.dot(pooled, w_ref[...],
                             preferred_element_type=jnp.float32) + fb_ref[...]
    return pl.pallas_call(
        body,
        out_shape=jax.ShapeDtypeStruct((NG, 1), jnp.float32),
    )(agg, deg, b2, batchq, fc_W, fc_b)


def kernel(x, edge_index, batch, W1, b1, W2, b2, fc_W, fc_b):
    ei = jnp.transpose(
        edge_index.astype(jnp.int32).reshape(2, NCHUNK, K), (1, 0, 2))
    ei = jnp.pad(ei, ((0, NW * CH - NCHUNK), (0, 0), (0, 0)),
                 constant_values=NN)
    batchq = jnp.transpose(jnp.concatenate(
        [batch.astype(jnp.int32),
         jnp.full((PADN - NN,), -1, jnp.int32)]).reshape(PADR, 8))
    b1r = b1.reshape(1, DH)
    b2r = b2.reshape(1, DH)
    fbr = fc_b.reshape(1, 1)

    y1 = _tc_in(x, W1)
    agg1, deg = _sc_agg(ei, y1, compute_deg=True)
    agg1t = agg1.reshape(NC, PADR, 8 * DH)
    degt = deg.reshape(NC, PADR, 8 * DH)
    y2 = _tc_mid(agg1t, degt, b1r, W2).reshape(PADN, DH)
    (agg2,) = _sc_agg(ei, y2, compute_deg=False)
    out = _tc_out(agg2.reshape(NC, PADR, 8 * DH), degt, b2r, batchq,
                  fc_W, fbr)
    return out


# speedup vs baseline: 37.8680x; 1.0105x over previous
import functools

import jax
import jax.numpy as jnp
from jax import lax
from jax.experimental import pallas as pl
from jax.experimental.pallas import tpu as pltpu
from jax.experimental.pallas import tpu_sc as plsc

NN = 10000
NE = 320000
NG = 64
DI = 128
DH = 16

NC = 2
NS = 16
NW = NC * NS
K = 128
CH = 79
NCHUNK = NE // K
PADN = NN + 112
RPT = PADN // NS


def _sc_agg(ei, y, compute_deg):
    mesh = plsc.VectorSubcoreMesh(
        core_axis_name="c", subcore_axis_name="s", num_cores=NC,
        num_subcores=NS)

    out_type = [jax.ShapeDtypeStruct((NC, PADN, DH), jnp.float32)]
    scratch = {
        "ei_v": pltpu.VMEM((CH, 2, K), jnp.int32),
        "rows_v": pltpu.VMEM((2, K, DH), jnp.float32),
        "stage_v": pltpu.VMEM((RPT, DH), jnp.float32),
        "acc_sh": pltpu.VMEM_SHARED((PADN, DH), jnp.float32),
        "y_sh": pltpu.VMEM_SHARED((PADN, DH), jnp.float32),
        "sem": pltpu.SemaphoreType.DMA,
        "sem_s": pltpu.SemaphoreType.DMA,
    }
    if compute_deg:
        out_type.append(jax.ShapeDtypeStruct((NC, PADN, DH), jnp.float32))
        scratch["ones_v"] = pltpu.VMEM((K, DH), jnp.float32)
        scratch["deg_sh"] = pltpu.VMEM_SHARED((PADN, DH), jnp.float32)

    def body(ei_hbm, y_hbm, *rest_in, ei_v, rows_v, stage_v, acc_sh, y_sh,
             sem, sem_s, ones_v=None, deg_sh=None):
        if compute_deg:
            ones_hbm, *outs = rest_in
        else:
            outs = rest_in
        if compute_deg:
            acc_out, deg_out = outs
        else:
            (acc_out,) = outs
        c = lax.axis_index("c")
        s = lax.axis_index("s")
        wid = c * NS + s

        def zrow(i, _):
            stage_v[i, :] = jnp.zeros((DH,), jnp.float32)
            return 0
        lax.fori_loop(0, RPT, zrow, 0)
        pltpu.sync_copy(stage_v, acc_sh.at[pl.ds(s * RPT, RPT)])
        if compute_deg:
            pltpu.sync_copy(stage_v, deg_sh.at[pl.ds(s * RPT, RPT)])
            pltpu.sync_copy(ones_hbm, ones_v)

        pltpu.sync_copy(ei_hbm.at[pl.ds(wid * CH, CH)], ei_v)

        @pl.when(s == 0)
        def _stage_y():
            pltpu.sync_copy(y_hbm, y_sh.at[pl.ds(0, y_hbm.shape[0])])

        plsc.subcore_barrier()

        def _wait_sc(j_idx):
            pltpu.make_async_copy(
                rows_v.at[0], acc_sh.at[ei_v.at[j_idx, 1]], sem_s).wait()
            if compute_deg:
                pltpu.make_async_copy(
                    ones_v, deg_sh.at[ei_v.at[j_idx, 1]], sem_s).wait()

        pltpu.async_copy(y_sh.at[ei_v.at[0, 0]], rows_v.at[0], sem)

        def chunk(j, _):
            pltpu.make_async_copy(
                y_sh.at[ei_v.at[0, 0]], rows_v.at[0], sem).wait()
            pltpu.async_copy(rows_v.at[lax.rem(j, 2)],
                             acc_sh.at[ei_v.at[j, 1]], sem_s, add=True)
            if compute_deg:
                pltpu.async_copy(ones_v, deg_sh.at[ei_v.at[j, 1]],
                                 sem_s, add=True)

            @pl.when(j + 1 < CH)
            def _prefetch():
                @pl.when(j >= 1)
                def _drain_prev():
                    _wait_sc(j - 1)
                pltpu.async_copy(
                    y_sh.at[ei_v.at[j + 1, 0]],
                    rows_v.at[lax.rem(j + 1, 2)], sem)
            return 0
        lax.fori_loop(0, CH, chunk, 0)
        _wait_sc(CH - 2)
        _wait_sc(CH - 1)

        plsc.subcore_barrier()
        sl = pl.ds(s * RPT, RPT)
        pltpu.sync_copy(acc_sh.at[sl], acc_out.at[c, sl])
        if compute_deg:
            pltpu.sync_copy(deg_sh.at[sl], deg_out.at[c, sl])

    f = pl.kernel(body, out_type=out_type, mesh=mesh,
                  scratch_types=scratch,
                  compiler_params=pltpu.CompilerParams(
                      use_tc_tiling_on_sc=False))
    if compute_deg:
        return f(ei, y, jnp.ones((K, DH), jnp.float32))
    return f(ei, y)


def _blockdiag(w_ref):
    w8 = jnp.concatenate([w_ref[...]] * 8, axis=0)
    w88 = jnp.concatenate([w8] * 8, axis=1)
    r = lax.broadcasted_iota(jnp.int32, (8 * DH, 8 * DH), 0) // DH
    c = lax.broadcasted_iota(jnp.int32, (8 * DH, 8 * DH), 1) // DH
    return jnp.where(r == c, w88, 0.0)


def _tile_bias(b_ref):
    return jnp.concatenate([b_ref[...]] * 8, axis=1)


def _tc_in(x, W1):
    def body(x_ref, w_ref, o_ref):
        x3 = jnp.reshape(x_ref[...], (NN // 8, 8, DI))
        w = w_ref[...]
        parts = []
        for q in range(8):
            xq = jnp.reshape(x3[:, q:q + 1, :], (NN // 8, DI))
            parts.append(jnp.dot(xq, w,
                                 preferred_element_type=jnp.float32))
        o_ref[...] = jnp.concatenate(parts, axis=1)
    return pl.pallas_call(
        body,
        out_shape=jax.ShapeDtypeStruct((NN // 8, 8 * DH), jnp.float32),
    )(x, W1)


PADR = PADN // 8


def _tc_mid(agg, deg, b1, W2):
    def body(a_ref, d_ref, b_ref, w_ref, o_ref):
        dg = jnp.maximum(d_ref[0] + d_ref[1], 1.0)
        h = (a_ref[0] + a_ref[1]) / dg + _tile_bias(b_ref)
        h = jnp.maximum(h, 0.0)
        o_ref[...] = jnp.dot(h, _blockdiag(w_ref),
                             preferred_element_type=jnp.float32)
    return pl.pallas_call(
        body,
        out_shape=jax.ShapeDtypeStruct((PADR, 8 * DH), jnp.float32),
    )(agg, deg, b1, W2)


def _tc_out(agg, deg, b2, batchq, fc_W, fc_b):
    def body(a_ref, d_ref, b_ref, bat_ref, w_ref, fb_ref, o_ref):
        dg = jnp.maximum(d_ref[0] + d_ref[1], 1.0)
        h = (a_ref[0] + a_ref[1]) / dg + _tile_bias(b_ref)
        h = jnp.maximum(h, 0.0)
        gids = lax.broadcasted_iota(jnp.int32, (NG, PADR), 0)
        sums = jnp.zeros((NG, DH), jnp.float32)
        cnts = jnp.zeros((NG, DH), jnp.float32)
        ones = jnp.ones((PADR, DH), jnp.float32)
        for q in range(8):
            oh = jnp.where(bat_ref[q:q + 1, :] == gids, 1.0, 0.0)
            hq = h[:, q * DH:(q + 1) * DH]
            sums = sums + lax.dot_general(
                oh, hq, (((1,), (0,)), ((), ())),
                preferred_element_type=jnp.float32)
            cnts = cnts + lax.dot_general(
                oh, ones, (((1,), (0,)), ((), ())),
                preferred_element_type=jnp.float32)
        pooled = sums / jnp.maximum(cnts, 1.0)
        o_ref[...] = jnp.dot(pooled, w_ref[...],
                             preferred_element_type=jnp.float32) + fb_ref[...]
    return pl.pallas_call(
        body,
        out_shape=jax.ShapeDtypeStruct((NG, 1), jnp.float32),
    )(agg, deg, b2, batchq, fc_W, fc_b)


def kernel(x, edge_index, batch, W1, b1, W2, b2, fc_W, fc_b):
    ei = jnp.transpose(
        edge_index.astype(jnp.int32).reshape(2, NCHUNK, K), (1, 0, 2))
    ei = jnp.pad(ei, ((0, NW * CH - NCHUNK), (0, 0), (0, 0)),
                 constant_values=NN)
    batchq = jnp.transpose(jnp.concatenate(
        [batch.astype(jnp.int32),
         jnp.full((PADN - NN,), -1, jnp.int32)]).reshape(PADR, 8))
    b1r = b1.reshape(1, DH)
    b2r = b2.reshape(1, DH)
    fbr = fc_b.reshape(1, 1)

    y1 = _tc_in(x, W1).reshape(NN, DH)
    agg1, deg = _sc_agg(ei, y1, compute_deg=True)
    agg1t = agg1.reshape(NC, PADR, 8 * DH)
    degt = deg.reshape(NC, PADR, 8 * DH)
    y2 = _tc_mid(agg1t, degt, b1r, W2).reshape(PADN, DH)
    (agg2,) = _sc_agg(ei, y2, compute_deg=False)
    out = _tc_out(agg2.reshape(NC, PADR, 8 * DH), degt, b2r, batchq,
                  fc_W, fbr)
    return out


# speedup vs baseline: 37.8941x; 1.0007x over previous
import jax
import jax.numpy as jnp
from jax import lax
from jax.experimental import pallas as pl
from jax.experimental.pallas import tpu as pltpu
from jax.experimental.pallas import tpu_sc as plsc

NN = 10000
NE = 320000
NG = 64
DI = 128
DH = 16

NC = 2
NS = 16
NW = NC * NS
K = 128
CH = 79
NCHUNK = NE // K
PADN = NN + 112
RPT = PADN // NS


def _sc_agg(ei, y, compute_deg):
    mesh = plsc.VectorSubcoreMesh(
        core_axis_name="c", subcore_axis_name="s", num_cores=NC,
        num_subcores=NS)

    out_type = [jax.ShapeDtypeStruct((NC, PADN, DH), jnp.float32)]
    scratch = {
        "ei_v": pltpu.VMEM((CH, 2, K), jnp.int32),
        "rows_v": pltpu.VMEM((2, K, DH), jnp.float32),
        "stage_v": pltpu.VMEM((RPT, DH), jnp.float32),
        "acc_sh": pltpu.VMEM_SHARED((PADN, DH), jnp.float32),
        "y_sh": pltpu.VMEM_SHARED((PADN, DH), jnp.float32),
        "sem": pltpu.SemaphoreType.DMA,
        "sem_s": pltpu.SemaphoreType.DMA,
    }
    if compute_deg:
        out_type.append(jax.ShapeDtypeStruct((NC, PADN, DH), jnp.float32))
        scratch["ones_v"] = pltpu.VMEM((K, DH), jnp.float32)
        scratch["deg_sh"] = pltpu.VMEM_SHARED((PADN, DH), jnp.float32)

    def body(ei_hbm, y_hbm, *rest_in, ei_v, rows_v, stage_v, acc_sh, y_sh,
             sem, sem_s, ones_v=None, deg_sh=None):
        if compute_deg:
            ones_hbm, *outs = rest_in
        else:
            outs = rest_in
        if compute_deg:
            acc_out, deg_out = outs
        else:
            (acc_out,) = outs
        c = lax.axis_index("c")
        s = lax.axis_index("s")
        wid = c * NS + s

        def zrow(i, _):
            stage_v[i, :] = jnp.zeros((DH,), jnp.float32)
            return 0
        lax.fori_loop(0, RPT, zrow, 0)
        pltpu.sync_copy(stage_v, acc_sh.at[pl.ds(s * RPT, RPT)])
        if compute_deg:
            pltpu.sync_copy(stage_v, deg_sh.at[pl.ds(s * RPT, RPT)])
            pltpu.sync_copy(ones_hbm, ones_v)

        pltpu.sync_copy(ei_hbm.at[pl.ds(wid * CH, CH)], ei_v)

        @pl.when(s == 0)
        def _stage_y():
            pltpu.sync_copy(y_hbm, y_sh.at[pl.ds(0, y_hbm.shape[0])])

        plsc.subcore_barrier()

        def _wait_sc(j_idx):
            pltpu.make_async_copy(
                rows_v.at[0], acc_sh.at[ei_v.at[j_idx, 1]], sem_s).wait()
            if compute_deg:
                pltpu.make_async_copy(
                    ones_v, deg_sh.at[ei_v.at[j_idx, 1]], sem_s).wait()

        pltpu.async_copy(y_sh.at[ei_v.at[0, 0]], rows_v.at[0], sem)

        def chunk(j, _):
            pltpu.make_async_copy(
                y_sh.at[ei_v.at[0, 0]], rows_v.at[0], sem).wait()
            pltpu.async_copy(rows_v.at[lax.rem(j, 2)],
                             acc_sh.at[ei_v.at[j, 1]], sem_s, add=True)
            if compute_deg:
                pltpu.async_copy(ones_v, deg_sh.at[ei_v.at[j, 1]],
                                 sem_s, add=True)

            @pl.when(j + 1 < CH)
            def _prefetch():
                @pl.when(j >= 1)
                def _drain_prev():
                    _wait_sc(j - 1)
                pltpu.async_copy(
                    y_sh.at[ei_v.at[j + 1, 0]],
                    rows_v.at[lax.rem(j + 1, 2)], sem)
            return 0
        lax.fori_loop(0, CH, chunk, 0)
        _wait_sc(CH - 2)
        _wait_sc(CH - 1)

        plsc.subcore_barrier()
        sl = pl.ds(s * RPT, RPT)
        pltpu.sync_copy(acc_sh.at[sl], acc_out.at[c, sl])
        if compute_deg:
            pltpu.sync_copy(deg_sh.at[sl], deg_out.at[c, sl])

    f = pl.kernel(body, out_type=out_type, mesh=mesh,
                  scratch_types=scratch,
                  compiler_params=pltpu.CompilerParams(
                      use_tc_tiling_on_sc=False))
    if compute_deg:
        return f(ei, y, jnp.ones((K, DH), jnp.float32))
    return f(ei, y)


def _blockdiag(w_ref):
    w8 = jnp.concatenate([w_ref[...]] * 8, axis=0)
    w88 = jnp.concatenate([w8] * 8, axis=1)
    r = lax.broadcasted_iota(jnp.int32, (8 * DH, 8 * DH), 0) // DH
    c = lax.broadcasted_iota(jnp.int32, (8 * DH, 8 * DH), 1) // DH
    return jnp.where(r == c, w88, 0.0)


def _tile_bias(b_ref):
    return jnp.concatenate([b_ref[...]] * 8, axis=1)


def _tc_in(x, W1):
    def body(x_ref, w_ref, o_ref):
        x3 = jnp.reshape(x_ref[...], (NN // 8, 8, DI))
        w = w_ref[...]
        parts = []
        for q in range(8):
            xq = jnp.reshape(x3[:, q:q + 1, :], (NN // 8, DI))
            parts.append(jnp.dot(xq, w,
                                 preferred_element_type=jnp.float32))
        o_ref[...] = jnp.concatenate(parts, axis=1)
    return pl.pallas_call(
        body,
        out_shape=jax.ShapeDtypeStruct((NN // 8, 8 * DH), jnp.float32),
    )(x, W1)


PADR = PADN // 8


def _tc_mid(agg, deg, b1, W2):
    def body(a_ref, d_ref, b_ref, w_ref, o_ref):
        dg = jnp.maximum(d_ref[0] + d_ref[1], 1.0)
        h = (a_ref[0] + a_ref[1]) / dg + _tile_bias(b_ref)
        h = jnp.maximum(h, 0.0)
        o_ref[...] = jnp.dot(h, _blockdiag(w_ref),
                             preferred_element_type=jnp.float32)
    return pl.pallas_call(
        body,
        out_shape=jax.ShapeDtypeStruct((PADR, 8 * DH), jnp.float32),
    )(agg, deg, b1, W2)


def _tc_out(agg, deg, b2, batchq, fc_W, fc_b):
    def body(a_ref, d_ref, b_ref, bat_ref, w_ref, fb_ref, o_ref):
        dg = jnp.maximum(d_ref[0] + d_ref[1], 1.0)
        h = (a_ref[0] + a_ref[1]) / dg + _tile_bias(b_ref)
        h = jnp.maximum(h, 0.0)
        gids = lax.broadcasted_iota(jnp.int32, (NG, PADR), 0)
        sums = jnp.zeros((NG, DH), jnp.float32)
        cnts = jnp.zeros((NG, DH), jnp.float32)
        ones = jnp.ones((PADR, DH), jnp.float32)
        for q in range(8):
            oh = jnp.where(bat_ref[q:q + 1, :] == gids, 1.0, 0.0)
            hq = h[:, q * DH:(q + 1) * DH]
            sums = sums + lax.dot_general(
                oh, hq, (((1,), (0,)), ((), ())),
                preferred_element_type=jnp.float32)
            cnts = cnts + lax.dot_general(
                oh, ones, (((1,), (0,)), ((), ())),
                preferred_element_type=jnp.float32)
        pooled = sums / jnp.maximum(cnts, 1.0)
        o_ref[...] = jnp.dot(pooled, w_ref[...],
                             preferred_element_type=jnp.float32) + fb_ref[...]
    return pl.pallas_call(
        body,
        out_shape=jax.ShapeDtypeStruct((NG, 1), jnp.float32),
    )(agg, deg, b2, batchq, fc_W, fc_b)


def kernel(x, edge_index, batch, W1, b1, W2, b2, fc_W, fc_b):
    ei = jnp.transpose(
        edge_index.astype(jnp.int32).reshape(2, NCHUNK, K), (1, 0, 2))
    ei = jnp.pad(ei, ((0, NW * CH - NCHUNK), (0, 0), (0, 0)),
                 constant_values=NN)
    batchq = jnp.transpose(jnp.concatenate(
        [batch.astype(jnp.int32),
         jnp.full((PADN - NN,), -1, jnp.int32)]).reshape(PADR, 8))
    b1r = b1.reshape(1, DH)
    b2r = b2.reshape(1, DH)
    fbr = fc_b.reshape(1, 1)

    y1 = _tc_in(x, W1).reshape(NN, DH)
    agg1, deg = _sc_agg(ei, y1, compute_deg=True)
    agg1t = agg1.reshape(NC, PADR, 8 * DH)
    degt = deg.reshape(NC, PADR, 8 * DH)
    y2 = _tc_mid(agg1t, degt, b1r, W2).reshape(PADN, DH)
    (agg2,) = _sc_agg(ei, y2, compute_deg=False)
    out = _tc_out(agg2.reshape(NC, PADR, 8 * DH), degt, b2r, batchq,
                  fc_W, fbr)
    return out
